# baseline (speedup 1.0000x reference)
import functools

import jax
import jax.numpy as jnp
from jax import lax
from jax.experimental import pallas as pl
from jax.experimental.pallas import tpu as pltpu
from jax.experimental.pallas import tpu_sc as plsc

L = 16
ROW = 128
SEG = 1536


def _build(B, NF, FE, EMB, n_fu, n_fi, n_ue, n_ie, n_ub, n_ib):
    NC, NS = 2, 16
    NW = NC * NS
    PW = B // NW
    C = 16
    R = PW // C
    NR = C * NF
    CH = 4
    CHN = NR // CH
    assert NR == CH * CHN and CHN % 8 == 0 and CHN <= 128

    mesh = plsc.VectorSubcoreMesh(
        core_axis_name="c", subcore_axis_name="s",
        num_cores=NC, num_subcores=NS)

    @functools.partial(
        pl.kernel,
        out_type=jax.ShapeDtypeStruct((B,), jnp.float32),
        mesh=mesh,
        compiler_params=pltpu.CompilerParams(
            needs_layout_passes=False, use_tc_tiling_on_sc=True),
        scratch_types=[
            pltpu.VMEM((SEG,), jnp.int32),
            pltpu.VMEM((NR, ROW), jnp.float32),
            pltpu.VMEM((NR, ROW), jnp.float32),
            pltpu.VMEM((C, ROW), jnp.float32),
            pltpu.VMEM((C, ROW), jnp.float32),
            pltpu.VMEM((C, ROW), jnp.float32),
            pltpu.VMEM((C, ROW), jnp.float32),
            pltpu.VMEM((ROW,), jnp.float32),
            pltpu.VMEM((PW,), jnp.float32),
            pltpu.SemaphoreType.DMA,
        ],
    )
    def mf_kernel(fu_tab, fi_tab, ue_tab, ie_tab, ub_tab, ib_tab,
                  meta_hbm, mean_hbm, out,
                  meta_v, fu_rows, fi_rows, u_rows, i_rows, ub_rows,
                  ib_rows, mean_v, out_v, sem):
        wid = lax.axis_index("s") * NC + lax.axis_index("c")
        pltpu.sync_copy(mean_hbm, mean_v)
        lanes = lax.iota(jnp.int32, L)

        def round_body(r, carry):
            pltpu.sync_copy(
                meta_hbm.at[pl.ds((wid * R + r) * SEG, SEG)], meta_v)
            cps = []
            for c in range(CH):
                cps.append(pltpu.async_copy(
                    fu_tab.at[meta_v.at[pl.ds(c * CHN, CHN)]],
                    fu_rows.at[pl.ds(c * CHN, CHN)], sem))
                cps.append(pltpu.async_copy(
                    fi_tab.at[meta_v.at[pl.ds(NR + c * CHN, CHN)]],
                    fi_rows.at[pl.ds(c * CHN, CHN)], sem))
            cps.append(pltpu.async_copy(
                ue_tab.at[meta_v.at[pl.ds(3 * NR + 1 * C, C)]], u_rows, sem))
            cps.append(pltpu.async_copy(
                ie_tab.at[meta_v.at[pl.ds(3 * NR + 3 * C, C)]], i_rows, sem))
            cps.append(pltpu.async_copy(
                ub_tab.at[meta_v.at[pl.ds(3 * NR + 5 * C, C)]], ub_rows, sem))
            cps.append(pltpu.async_copy(
                ib_tab.at[meta_v.at[pl.ds(3 * NR + 7 * C, C)]], ib_rows, sem))
            for cp in cps:
                cp.wait()

            fi_off = meta_v[pl.ds(3 * NR + 0 * C, C)]
            u_off = meta_v[pl.ds(3 * NR + 2 * C, C)]
            i_off = meta_v[pl.ds(3 * NR + 4 * C, C)]
            ub_off = meta_v[pl.ds(3 * NR + 6 * C, C)]
            ib_off = meta_v[pl.ds(3 * NR + 8 * C, C)]

            acc = (plsc.load_gather(ub_rows, [lanes, ub_off])
                   + plsc.load_gather(ib_rows, [lanes, ib_off])
                   + mean_v[pl.ds(0, L)])
            for d in range(EMB):
                acc = acc + (plsc.load_gather(u_rows, [lanes, u_off + d])
                             * plsc.load_gather(i_rows, [lanes, i_off + d]))

            def feat_body(j, a):
                rows = lanes + j * C
                fu_off = meta_v[pl.ds(2 * NR + j * C, C)]
                for d in range(FE):
                    a = a + (plsc.load_gather(fu_rows, [rows, fu_off + d])
                             * plsc.load_gather(fi_rows, [rows, fi_off + d]))
                return a

            acc = lax.fori_loop(0, NF, feat_body, acc)
            out_v[pl.ds(r * C, C)] = acc
            return carry

        lax.fori_loop(0, R, round_body, 0)
        pltpu.sync_copy(out_v, out.at[pl.ds(wid * PW, PW)])

    return mf_kernel


def kernel(u_id, i_id, features, user_emb, user_bias, item_emb, item_bias,
           feat_u, feat_i, mean):
    B = u_id.shape[0]
    NF = features.shape[1]
    FV, FE = feat_u.shape[1], feat_u.shape[2]
    NI = feat_i.shape[1]
    NU, EMB = user_emb.shape
    NI2 = item_emb.shape[0]
    i32 = jnp.int32
    NW, C = 32, 16
    PW = B // NW
    R = PW // C
    NR = C * NF

    fu_tab = feat_u.reshape(NF * FV * FE // ROW, ROW)
    fi_tab = feat_i.reshape(NF * NI * FE // ROW, ROW)
    ue_tab = user_emb.reshape(NU * EMB // ROW, ROW)
    ie_tab = item_emb.reshape(NI2 * EMB // ROW, ROW)
    ub_pad = (-NU) % ROW
    ib_pad = (-NI2) % ROW
    ub_tab = jnp.pad(user_bias.reshape(-1), (0, ub_pad)).reshape(-1, ROW)
    ib_tab = jnp.pad(item_bias.reshape(-1), (0, ib_pad)).reshape(-1, ROW)
    mean128 = jnp.broadcast_to(mean.astype(jnp.float32), (ROW,))

    feats = features.astype(i32)
    iid = i_id.astype(i32)
    uid = u_id.astype(i32)
    f_ar = jnp.arange(NF, dtype=i32)[None, :]
    fu_row = feats // 4 + f_ar * (FV * FE // ROW)
    fu_off = (feats % 4) * FE
    fi_row = iid[:, None] // 4 + f_ar * (NI * FE // ROW)

    def seg2(x):
        return (x.reshape(NW, R, C, NF).transpose(0, 1, 3, 2)
                .reshape(NW, R, NF * C))

    def seg1(x):
        return x.reshape(NW, R, C)

    upr = ROW // EMB
    parts = [
        seg2(fu_row), seg2(fi_row), seg2(fu_off),
        seg1((iid % 4) * FE),
        seg1(uid // upr), seg1((uid % upr) * EMB),
        seg1(iid // upr), seg1((iid % upr) * EMB),
        seg1(uid // ROW), seg1(uid % ROW),
        seg1(iid // ROW), seg1(iid % ROW),
        jnp.zeros((NW, R, SEG - 3 * NR - 9 * C), i32),
    ]
    meta = jnp.concatenate(parts, axis=-1).reshape(-1)

    fn = _build(B, NF, FE, EMB, *(t.shape[0] for t in
                                  (fu_tab, fi_tab, ue_tab, ie_tab,
                                   ub_tab, ib_tab)))
    return fn(fu_tab, fi_tab, ue_tab, ie_tab, ub_tab, ib_tab, meta, mean128)
